# baseline (speedup 1.0000x reference)
import functools
import jax
import jax.numpy as jnp
from jax import lax
from jax.experimental import pallas as pl
from jax.experimental.pallas import tpu as pltpu
from jax.experimental.pallas import tpu_sc as plsc

HIDDEN = 2048
NUM_EXPERTS = 8
ROWS_BLK = 1024
N_TOK = 8192


def _stats_kernel(x_ref, wr_ref, br_ref, lt_ref, s_ref):
    i = pl.program_id(0)
    xb = x_ref[...]
    logits = jax.lax.dot_general(
        xb, wr_ref[...], (((1,), (0,)), ((), ())),
        preferred_element_type=jnp.float32,
    ) + br_ref[...]
    lt_ref[...] = jnp.transpose(logits)

    part = jnp.sum(xb, axis=0, keepdims=True)

    @pl.when(i == 0)
    def _():
        s_ref[...] = part

    @pl.when(i != 0)
    def _():
        s_ref[...] += part


def _make_sc_router():
    info = plsc.get_sparse_core_info()
    nc, ns, nl = info.num_cores, info.num_subcores, info.num_lanes
    nw = nc * ns
    tok_per_w = N_TOK // nw

    mesh = plsc.VectorSubcoreMesh(core_axis_name="c", subcore_axis_name="s")

    @functools.partial(
        pl.kernel, mesh=mesh,
        out_type=jax.ShapeDtypeStruct((N_TOK,), jnp.float32),
        scratch_types=[
            pltpu.VMEM((NUM_EXPERTS, tok_per_w), jnp.float32),
            pltpu.VMEM((tok_per_w,), jnp.float32),
        ],
    )
    def sc_router(lt_hbm, g_hbm, lt_v, g_v):
        wid = lax.axis_index("s") * nc + lax.axis_index("c")
        base = wid * tok_per_w
        pltpu.sync_copy(lt_hbm.at[:, pl.ds(base, tok_per_w)], lt_v)
        for c in range(tok_per_w // nl):
            sl = pl.ds(c * nl, nl)
            vs = [lt_v[e, sl] for e in range(NUM_EXPERTS)]
            m1 = vs[0]
            for e in range(1, NUM_EXPERTS):
                m1 = jnp.maximum(m1, vs[e])
            l2 = jnp.full((nl,), -1e30, jnp.float32)
            free = jnp.full((nl,), 1.0, jnp.float32)
            den = jnp.zeros((nl,), jnp.float32)
            for e in range(NUM_EXPERTS):
                d = vs[e] - m1
                den = den + jnp.exp(d)
                eqf = jnp.where(d >= 0.0, 1.0, 0.0)
                is_first = eqf * free
                free = free - is_first
                l2 = jnp.maximum(l2, d - is_first * 1e30)
            g_v[sl] = (1.0 + jnp.exp(l2)) / den
        pltpu.sync_copy(g_v, g_hbm.at[pl.ds(base, tok_per_w)])

    return sc_router


_sc_router = _make_sc_router()


def _write_kernel(g_ref, we_ref, be_ref, s_ref, y_ref, v_scr):
    i = pl.program_id(0)

    @pl.when(i == 0)
    def _():
        s_col = jnp.transpose(s_ref[...])
        v_scr[...] = jnp.sum(we_ref[...] * s_col, axis=0, keepdims=True) \
            + float(NUM_EXPERTS) * be_ref[...]

    y_ref[...] = g_ref[...] * v_scr[...]


def kernel(x, Wr, br, We, be):
    b, seq, h = x.shape
    n = b * seq
    xf = x.reshape(n, h)
    nblk = n // ROWS_BLK

    lt, s = pl.pallas_call(
        _stats_kernel,
        grid=(nblk,),
        in_specs=[
            pl.BlockSpec((ROWS_BLK, h), lambda i: (i, 0)),
            pl.BlockSpec((h, NUM_EXPERTS), lambda i: (0, 0)),
            pl.BlockSpec((1, NUM_EXPERTS), lambda i: (0, 0)),
        ],
        out_specs=[
            pl.BlockSpec((NUM_EXPERTS, ROWS_BLK), lambda i: (0, i)),
            pl.BlockSpec((1, h), lambda i: (0, 0)),
        ],
        out_shape=[
            jax.ShapeDtypeStruct((NUM_EXPERTS, n), jnp.float32),
            jax.ShapeDtypeStruct((1, h), jnp.float32),
        ],
    )(xf, Wr, br.reshape(1, NUM_EXPERTS))

    g = _sc_router(lt).reshape(n, 1)

    y = pl.pallas_call(
        _write_kernel,
        grid=(nblk,),
        in_specs=[
            pl.BlockSpec((ROWS_BLK, 1), lambda i: (i, 0)),
            pl.BlockSpec((h, h), lambda i: (0, 0)),
            pl.BlockSpec((1, h), lambda i: (0, 0)),
            pl.BlockSpec((1, h), lambda i: (0, 0)),
        ],
        out_specs=pl.BlockSpec((ROWS_BLK, h), lambda i: (i, 0)),
        out_shape=jax.ShapeDtypeStruct((n, h), jnp.float32),
        scratch_shapes=[pltpu.VMEM((1, h), jnp.float32)],
    )(g, We, be.reshape(1, h), s)

    return y.reshape(b, seq, h)


# speedup vs baseline: 1.4407x; 1.4407x over previous
import jax
import jax.numpy as jnp
from jax.experimental import pallas as pl
from jax.experimental.pallas import tpu as pltpu

HIDDEN = 2048
NUM_EXPERTS = 8
ROWS_BLK = 1024


def _fused_kernel(x_ref, wr_ref, br_ref, we_ref, be_ref, y_ref,
                  g_scr, s_scr, v_scr):
    p = pl.program_id(0)
    i = pl.program_id(1)

    @pl.when(p == 0)
    def _():
        xb = x_ref[...]
        logits = jax.lax.dot_general(
            xb, wr_ref[...], (((1,), (0,)), ((), ())),
            preferred_element_type=jnp.float32,
        ) + br_ref[...]
        m1 = jnp.max(logits, axis=-1, keepdims=True)
        am = jnp.argmax(logits, axis=-1)[:, None]
        col = jax.lax.broadcasted_iota(jnp.int32, logits.shape, 1)
        l2 = jnp.max(jnp.where(col == am, -jnp.inf, logits), axis=-1,
                     keepdims=True)
        denom = jnp.sum(jnp.exp(logits - m1), axis=-1, keepdims=True)
        g_scr[pl.ds(i * ROWS_BLK, ROWS_BLK), :] = \
            (1.0 + jnp.exp(l2 - m1)) / denom

        part = jnp.sum(xb, axis=0, keepdims=True)

        @pl.when(i == 0)
        def _():
            s_scr[...] = part

        @pl.when(i != 0)
        def _():
            s_scr[...] += part

    @pl.when((p == 1) & (i == 0))
    def _():
        s_col = jnp.transpose(s_scr[...])
        v_scr[...] = jnp.sum(we_ref[...] * s_col, axis=0, keepdims=True) \
            + float(NUM_EXPERTS) * be_ref[...]

    @pl.when(p == 1)
    def _():
        y_ref[...] = g_scr[pl.ds(i * ROWS_BLK, ROWS_BLK), :] * v_scr[...]


def kernel(x, Wr, br, We, be):
    b, seq, h = x.shape
    n = b * seq
    xf = x.reshape(n, h)
    nblk = n // ROWS_BLK

    y = pl.pallas_call(
        _fused_kernel,
        grid=(2, nblk),
        in_specs=[
            pl.BlockSpec((ROWS_BLK, h),
                         lambda p, i: (jnp.where(p == 0, i, nblk - 1), 0)),
            pl.BlockSpec((h, NUM_EXPERTS), lambda p, i: (0, 0)),
            pl.BlockSpec((1, NUM_EXPERTS), lambda p, i: (0, 0)),
            pl.BlockSpec((h, h), lambda p, i: (0, 0)),
            pl.BlockSpec((1, h), lambda p, i: (0, 0)),
        ],
        out_specs=pl.BlockSpec((ROWS_BLK, h),
                               lambda p, i: (jnp.where(p == 0, 0, i), 0)),
        out_shape=jax.ShapeDtypeStruct((n, h), jnp.float32),
        scratch_shapes=[
            pltpu.VMEM((n, 1), jnp.float32),
            pltpu.VMEM((1, h), jnp.float32),
            pltpu.VMEM((1, h), jnp.float32),
        ],
    )(xf, Wr, br.reshape(1, NUM_EXPERTS), We, be.reshape(1, h))

    return y.reshape(b, seq, h)
